# baseline (speedup 1.0000x reference)
import dataclasses
import functools

import jax
import jax.numpy as jnp
from jax import lax
from jax.experimental import pallas as pl
from jax.experimental.pallas import tpu as pltpu
from jax.experimental.pallas import tpu_sc as plsc

_LANES = 16
_NC, _NS = 2, 16
_NW = _NC * _NS
_CHUNK = 48
_CB = 4
_CORE0_FRAC = 0.70


def _tc_pre_body(x_ref, w_ref, as_ref, ad_ref, h_ref, asrc_ref, adst_ref,
                 selfw_ref):
    h = jnp.dot(x_ref[...], w_ref[...], preferred_element_type=jnp.float32)
    h_ref[...] = h
    a_s = jnp.sum(h * as_ref[...], axis=1, keepdims=True)
    a_d = jnp.sum(h * ad_ref[...], axis=1, keepdims=True)
    asrc_ref[...] = a_s
    adst_ref[...] = a_d
    e = a_s + a_d
    selfw_ref[...] = jnp.exp(jnp.maximum(e, 0.2 * e))


def _tc_pre(x, W, att_s, att_d, blk=1000):
    n, d = x.shape
    return pl.pallas_call(
        _tc_pre_body,
        grid=(n // blk,),
        in_specs=[
            pl.BlockSpec((blk, d), lambda i: (i, 0)),
            pl.BlockSpec((d, d), lambda i: (0, 0)),
            pl.BlockSpec((1, d), lambda i: (0, 0)),
            pl.BlockSpec((1, d), lambda i: (0, 0)),
        ],
        out_specs=[
            pl.BlockSpec((blk, d), lambda i: (i, 0)),
            pl.BlockSpec((blk, 1), lambda i: (i, 0)),
            pl.BlockSpec((blk, 1), lambda i: (i, 0)),
            pl.BlockSpec((blk, 1), lambda i: (i, 0)),
        ],
        out_shape=[
            jax.ShapeDtypeStruct((n, d), jnp.float32),
            jax.ShapeDtypeStruct((n, 1), jnp.float32),
            jax.ShapeDtypeStruct((n, 1), jnp.float32),
            jax.ShapeDtypeStruct((n, 1), jnp.float32),
        ],
    )(x, W, att_s.reshape(1, d), att_d.reshape(1, d))


def _combine(s_ref, den_ref, h_ref, selfw_ref, b_ref):
    num = s_ref[0] + s_ref[1] + selfw_ref[...] * h_ref[...]
    den = jnp.sum(den_ref[...], axis=0) + selfw_ref[...] + 1e-16
    return jnp.maximum(num / den + b_ref[...], 0.0)


def _tc_mid_body(s_ref, den_ref, h_ref, selfw_ref, b_ref, w_ref, as_ref,
                 ad_ref, h2_ref, asrc_ref, adst_ref, selfw2_ref):
    x2 = _combine(s_ref, den_ref, h_ref, selfw_ref, b_ref)
    h2 = jnp.dot(x2, w_ref[...], preferred_element_type=jnp.float32)
    h2_ref[...] = h2
    a_s = jnp.sum(h2 * as_ref[...], axis=1, keepdims=True)
    a_d = jnp.sum(h2 * ad_ref[...], axis=1, keepdims=True)
    asrc_ref[...] = a_s
    adst_ref[...] = a_d
    e = a_s + a_d
    selfw2_ref[...] = jnp.exp(jnp.maximum(e, 0.2 * e))


def _tc_mid(S, den, h, selfw, b, W, att_s, att_d, blk=1000):
    n, d = h.shape
    return pl.pallas_call(
        _tc_mid_body,
        grid=(n // blk,),
        in_specs=[
            pl.BlockSpec((_NC, blk, d), lambda i: (0, i, 0)),
            pl.BlockSpec((_NW, blk, 1), lambda i: (0, i, 0)),
            pl.BlockSpec((blk, d), lambda i: (i, 0)),
            pl.BlockSpec((blk, 1), lambda i: (i, 0)),
            pl.BlockSpec((1, d), lambda i: (0, 0)),
            pl.BlockSpec((d, d), lambda i: (0, 0)),
            pl.BlockSpec((1, d), lambda i: (0, 0)),
            pl.BlockSpec((1, d), lambda i: (0, 0)),
        ],
        out_specs=[
            pl.BlockSpec((blk, d), lambda i: (i, 0)),
            pl.BlockSpec((blk, 1), lambda i: (i, 0)),
            pl.BlockSpec((blk, 1), lambda i: (i, 0)),
            pl.BlockSpec((blk, 1), lambda i: (i, 0)),
        ],
        out_shape=[
            jax.ShapeDtypeStruct((n, d), jnp.float32),
            jax.ShapeDtypeStruct((n, 1), jnp.float32),
            jax.ShapeDtypeStruct((n, 1), jnp.float32),
            jax.ShapeDtypeStruct((n, 1), jnp.float32),
        ],
    )(S, den, h, selfw, b.reshape(1, d), W, att_s.reshape(1, d),
      att_d.reshape(1, d))


def _tc_fin_body(s_ref, den_ref, h_ref, selfw_ref, b_ref, o_ref):
    o_ref[...] = _combine(s_ref, den_ref, h_ref, selfw_ref, b_ref)


def _tc_fin(S, den, h, selfw, b, blk=1000):
    n, d = h.shape
    return pl.pallas_call(
        _tc_fin_body,
        grid=(n // blk,),
        in_specs=[
            pl.BlockSpec((_NC, blk, d), lambda i: (0, i, 0)),
            pl.BlockSpec((_NW, blk, 1), lambda i: (0, i, 0)),
            pl.BlockSpec((blk, d), lambda i: (i, 0)),
            pl.BlockSpec((blk, 1), lambda i: (i, 0)),
            pl.BlockSpec((1, d), lambda i: (0, 0)),
        ],
        out_specs=pl.BlockSpec((blk, d), lambda i: (i, 0)),
        out_shape=jax.ShapeDtypeStruct((n, d), jnp.float32),
    )(S, den, h, selfw, b.reshape(1, d))


def _sc_edge_aggregate(h, asrc_p, adst_p, src2, dst2, srows, ct0, ct1):
    n, d = h.shape
    apad = asrc_p.shape[0]
    rpt = srows // _NS
    zfull, zrem = divmod(rpt, _CHUNK)
    mesh = plsc.VectorSubcoreMesh(core_axis_name="c", subcore_axis_name="s")
    cp = pltpu.CompilerParams()
    if "needs_layout_passes" in pltpu.CompilerParams.__dataclass_fields__:
        cp = dataclasses.replace(cp, needs_layout_passes=False)

    assert ct0 % _CB == 0 and ct1 % _CB == 0 and (_CB % 2) == 0
    assert ct0 >= 2 * _CB and ct1 >= 2 * _CB

    @functools.partial(
        pl.kernel,
        out_type=[
            jax.ShapeDtypeStruct((_NC, srows, d), jnp.float32),
            jax.ShapeDtypeStruct((_NW, apad), jnp.float32),
        ],
        mesh=mesh,
        compiler_params=cp,
        scratch_types=[
            pltpu.VMEM_SHARED((srows, d), jnp.float32),
            pltpu.VMEM((apad,), jnp.float32),
            pltpu.VMEM((apad,), jnp.float32),
            pltpu.VMEM((apad,), jnp.float32),
            pltpu.VMEM((2, _CB, _CHUNK), jnp.int32),
            pltpu.VMEM((2, _CB, _CHUNK), jnp.int32),
            pltpu.VMEM((_CHUNK,), jnp.float32),
            pltpu.VMEM((2, _CHUNK, d), jnp.float32),
            pltpu.SemaphoreType.DMA,
            pltpu.SemaphoreType.DMA,
            pltpu.SemaphoreType.DMA,
            pltpu.SemaphoreType.DMA,
            pltpu.SemaphoreType.DMA,
        ],
    )
    def sc_kernel(h_hbm, asrc_hbm, adst_hbm, src_hbm, dst_hbm,
                  num_hbm, den_hbm,
                  acc_sh, asrc_v, adst_v, denom_v, sidx_v, didx_v,
                  w_v, rows_v,
                  sem_g0, sem_g1, sem_s0, sem_s1, sem_i):
        c = lax.axis_index("c")
        s = lax.axis_index("s")
        wid = c * _NS + s
        ct = jnp.where(c == 0, ct0, ct1)
        nbm = ct // _CB
        row0 = jnp.where(c == 0, s * ct0, _NS * ct0 + s * ct1)
        sem_g = (sem_g0, sem_g1)
        sem_s = (sem_s0, sem_s1)
        pltpu.sync_copy(asrc_hbm, asrc_v)
        pltpu.sync_copy(adst_hbm, adst_v)

        zero = jnp.zeros((_LANES,), jnp.float32)
        lane = lax.iota(jnp.int32, _LANES)

        @pl.loop(0, apad // _LANES)
        def _(v):
            denom_v[pl.ds(v * _LANES, _LANES)] = zero

        @pl.loop(0, _CHUNK)
        def _(r):
            for k2 in range(d // _LANES):
                rows_v[0, r, pl.ds(k2 * _LANES, _LANES)] = zero

        @pl.loop(0, zfull)
        def _(z):
            base = s * rpt + z * _CHUNK
            pltpu.sync_copy(rows_v.at[0], acc_sh.at[pl.ds(base, _CHUNK)])

        if zrem:
            pltpu.sync_copy(
                rows_v.at[0, pl.ds(0, zrem)],
                acc_sh.at[pl.ds(s * rpt + zfull * _CHUNK, zrem)])

        plsc.subcore_barrier()

        def attention_weights(bsel, jj):
            for i in range(_CHUNK // _LANES):
                sidx = sidx_v[bsel, jj, pl.ds(i * _LANES, _LANES)]
                didx = didx_v[bsel, jj, pl.ds(i * _LANES, _LANES)]
                e = (plsc.load_gather(asrc_v, [sidx])
                     + plsc.load_gather(adst_v, [didx]))
                w = jnp.exp(jnp.maximum(e, e * 0.2))
                w_v[pl.ds(i * _LANES, _LANES)] = w
                dsort, wsort = plsc.sort_key_val(didx, w)
                cs = plsc.cumsum(wsort)
                prev_d = dsort.at[jnp.maximum(lane - 1, 0)].get(
                    mode="promise_in_bounds")
                next_d = dsort.at[jnp.minimum(lane + 1, _LANES - 1)].get(
                    mode="promise_in_bounds")
                is_head = (lane == 0) | (dsort != prev_d)
                is_tail = (lane == _LANES - 1) | (dsort != next_d)
                base = plsc.cummax(jnp.where(is_head, cs - wsort, -1.0))
                plsc.addupdate_scatter(denom_v, [dsort], cs - base,
                                       mask=is_tail)

        def chunk_body(jb, jj, par, la_bsel, la_jj, bsel):
            j = jb * _CB + jj
            rows_cur = rows_v.at[par]
            rows_nxt = rows_v.at[1 - par]

            @pl.when(j > 0)
            def _():
                pltpu.make_async_copy(h_hbm.at[pl.ds(0, _CHUNK)],
                                      rows_nxt, sem_s[1 - par]).wait()

            @pl.when(j + 1 < ct)
            def _():
                pltpu.async_copy(h_hbm.at[sidx_v.at[la_bsel, la_jj]],
                                 rows_nxt, sem_g[1 - par])

            attention_weights(bsel, jj)

            pltpu.make_async_copy(h_hbm.at[pl.ds(0, _CHUNK)],
                                  rows_cur, sem_g[par]).wait()

            @plsc.parallel_loop(0, _CHUNK, unroll=4)
            def _(r):
                wv = plsc.load_gather(w_v, [jnp.broadcast_to(r, (_LANES,))])
                for k2 in range(d // _LANES):
                    sl = pl.ds(k2 * _LANES, _LANES)
                    rows_cur[r, sl] = rows_cur[r, sl] * wv

            pltpu.async_copy(rows_cur, acc_sh.at[didx_v.at[bsel, jj]],
                             sem_s[par], add=True)

        def wait_idx_block():
            pltpu.make_async_copy(src_hbm.at[pl.ds(0, _CB)],
                                  sidx_v.at[0], sem_i).wait()
            pltpu.make_async_copy(dst_hbm.at[pl.ds(0, _CB)],
                                  didx_v.at[0], sem_i).wait()

        pltpu.sync_copy(src_hbm.at[pl.ds(row0, _CB)], sidx_v.at[0])
        pltpu.sync_copy(dst_hbm.at[pl.ds(row0, _CB)], didx_v.at[0])
        pltpu.async_copy(src_hbm.at[pl.ds(row0 + _CB, _CB)], sidx_v.at[1],
                         sem_i)
        pltpu.async_copy(dst_hbm.at[pl.ds(row0 + _CB, _CB)], didx_v.at[1],
                         sem_i)
        pltpu.async_copy(h_hbm.at[sidx_v.at[0, 0]], rows_v.at[0], sem_g0)

        @pl.loop(0, nbm)
        def _(jb):
            bsel = lax.rem(jb, 2)

            @pl.loop(0, (_CB - 2) // 2)
            def _(pp):
                chunk_body(jb, 2 * pp, 0, bsel, 2 * pp + 1, bsel)
                chunk_body(jb, 2 * pp + 1, 1, bsel, 2 * pp + 2, bsel)

            chunk_body(jb, _CB - 2, 0, bsel, _CB - 1, bsel)
            wait_idx_block()
            chunk_body(jb, _CB - 1, 1, 1 - bsel, 0, bsel)
            nxt = jnp.minimum(jb + 2, nbm - 1)
            pltpu.async_copy(src_hbm.at[pl.ds(row0 + nxt * _CB, _CB)],
                             sidx_v.at[bsel], sem_i)
            pltpu.async_copy(dst_hbm.at[pl.ds(row0 + nxt * _CB, _CB)],
                             didx_v.at[bsel], sem_i)

        pltpu.make_async_copy(h_hbm.at[pl.ds(0, _CHUNK)], rows_v.at[1],
                              sem_s1).wait()
        wait_idx_block()

        pltpu.sync_copy(denom_v, den_hbm.at[wid])
        plsc.subcore_barrier()

        pltpu.sync_copy(acc_sh.at[pl.ds(s * rpt, rpt)],
                        num_hbm.at[c, pl.ds(s * rpt, rpt)])

    return sc_kernel(h, asrc_p, adst_p, src2, dst2)


def kernel(x, edge_index, W1, as1, ad1, b1, W2, as2, ad2, b2):
    n, d = x.shape
    e = edge_index.shape[1]
    tpt = -(-e // (_NS * _CHUNK))
    tpt = ((tpt + _CB - 1) // _CB) * _CB
    ct0 = int(_CB * round(_CORE0_FRAC * tpt / _CB))
    ct0 = max(2 * _CB, min(tpt - 2 * _CB, ct0))
    ct1 = tpt - ct0
    ep = _NS * tpt * _CHUNK
    totch = _NS * tpt
    srows = ((n + 1 + _NS * _CHUNK - 1) // (_NS * _CHUNK)) * (_NS * _CHUNK)
    apad = ((n + 1 + _LANES - 1) // _LANES) * _LANES

    src = edge_index[0]
    dst = edge_index[1]
    srcp = jnp.concatenate(
        [src, jnp.zeros((ep - e,), jnp.int32)]).reshape(totch, _CHUNK)
    dstp = jnp.concatenate(
        [dst, jnp.full((ep - e,), n, jnp.int32)]).reshape(totch, _CHUNK)

    h1, a1s, a1d, sw1 = _tc_pre(x, W1, as1, ad1)
    a1sp = jnp.pad(a1s[:, 0], (0, apad - n))
    a1dp = jnp.pad(a1d[:, 0], (0, apad - n))
    S1, den1 = _sc_edge_aggregate(h1, a1sp, a1dp, srcp, dstp, srows, ct0, ct1)

    h2, a2s, a2d, sw2 = _tc_mid(S1, den1.reshape(_NW, apad, 1), h1, sw1, b1,
                                W2, as2, ad2)
    a2sp = jnp.pad(a2s[:, 0], (0, apad - n))
    a2dp = jnp.pad(a2d[:, 0], (0, apad - n))
    S2, den2 = _sc_edge_aggregate(h2, a2sp, a2dp, srcp, dstp, srows, ct0, ct1)

    return _tc_fin(S2, den2.reshape(_NW, apad, 1), h2, sw2, b2)


# speedup vs baseline: 1.0376x; 1.0376x over previous
import dataclasses
import functools

import jax
import jax.numpy as jnp
from jax import lax
from jax.experimental import pallas as pl
from jax.experimental.pallas import tpu as pltpu
from jax.experimental.pallas import tpu_sc as plsc

_LANES = 16
_NC, _NS = 2, 16
_NW = _NC * _NS
_CHUNK = 48
_CB = 4
_CORE0_FRAC = 0.65


def _tc_pre_body(x_ref, w_ref, as_ref, ad_ref, h_ref, asrc_ref, adst_ref,
                 selfw_ref):
    h = jnp.dot(x_ref[...], w_ref[...], preferred_element_type=jnp.float32)
    h_ref[...] = h
    a_s = jnp.sum(h * as_ref[...], axis=1, keepdims=True)
    a_d = jnp.sum(h * ad_ref[...], axis=1, keepdims=True)
    asrc_ref[...] = a_s
    adst_ref[...] = a_d
    e = a_s + a_d
    selfw_ref[...] = jnp.exp(jnp.maximum(e, 0.2 * e))


def _tc_pre(x, W, att_s, att_d, blk=1000):
    n, d = x.shape
    return pl.pallas_call(
        _tc_pre_body,
        grid=(n // blk,),
        in_specs=[
            pl.BlockSpec((blk, d), lambda i: (i, 0)),
            pl.BlockSpec((d, d), lambda i: (0, 0)),
            pl.BlockSpec((1, d), lambda i: (0, 0)),
            pl.BlockSpec((1, d), lambda i: (0, 0)),
        ],
        out_specs=[
            pl.BlockSpec((blk, d), lambda i: (i, 0)),
            pl.BlockSpec((blk, 1), lambda i: (i, 0)),
            pl.BlockSpec((blk, 1), lambda i: (i, 0)),
            pl.BlockSpec((blk, 1), lambda i: (i, 0)),
        ],
        out_shape=[
            jax.ShapeDtypeStruct((n, d), jnp.float32),
            jax.ShapeDtypeStruct((n, 1), jnp.float32),
            jax.ShapeDtypeStruct((n, 1), jnp.float32),
            jax.ShapeDtypeStruct((n, 1), jnp.float32),
        ],
    )(x, W, att_s.reshape(1, d), att_d.reshape(1, d))


def _combine(s_ref, den_ref, h_ref, selfw_ref, b_ref):
    num = s_ref[0] + s_ref[1] + selfw_ref[...] * h_ref[...]
    den = jnp.sum(den_ref[...], axis=0) + selfw_ref[...] + 1e-16
    return jnp.maximum(num / den + b_ref[...], 0.0)


def _tc_mid_body(s_ref, den_ref, h_ref, selfw_ref, b_ref, w_ref, as_ref,
                 ad_ref, h2_ref, asrc_ref, adst_ref, selfw2_ref):
    x2 = _combine(s_ref, den_ref, h_ref, selfw_ref, b_ref)
    h2 = jnp.dot(x2, w_ref[...], preferred_element_type=jnp.float32)
    h2_ref[...] = h2
    a_s = jnp.sum(h2 * as_ref[...], axis=1, keepdims=True)
    a_d = jnp.sum(h2 * ad_ref[...], axis=1, keepdims=True)
    asrc_ref[...] = a_s
    adst_ref[...] = a_d
    e = a_s + a_d
    selfw2_ref[...] = jnp.exp(jnp.maximum(e, 0.2 * e))


def _tc_mid(S, den, h, selfw, b, W, att_s, att_d, blk=1000):
    n, d = h.shape
    return pl.pallas_call(
        _tc_mid_body,
        grid=(n // blk,),
        in_specs=[
            pl.BlockSpec((_NC, blk, d), lambda i: (0, i, 0)),
            pl.BlockSpec((_NW, blk, 1), lambda i: (0, i, 0)),
            pl.BlockSpec((blk, d), lambda i: (i, 0)),
            pl.BlockSpec((blk, 1), lambda i: (i, 0)),
            pl.BlockSpec((1, d), lambda i: (0, 0)),
            pl.BlockSpec((d, d), lambda i: (0, 0)),
            pl.BlockSpec((1, d), lambda i: (0, 0)),
            pl.BlockSpec((1, d), lambda i: (0, 0)),
        ],
        out_specs=[
            pl.BlockSpec((blk, d), lambda i: (i, 0)),
            pl.BlockSpec((blk, 1), lambda i: (i, 0)),
            pl.BlockSpec((blk, 1), lambda i: (i, 0)),
            pl.BlockSpec((blk, 1), lambda i: (i, 0)),
        ],
        out_shape=[
            jax.ShapeDtypeStruct((n, d), jnp.float32),
            jax.ShapeDtypeStruct((n, 1), jnp.float32),
            jax.ShapeDtypeStruct((n, 1), jnp.float32),
            jax.ShapeDtypeStruct((n, 1), jnp.float32),
        ],
    )(S, den, h, selfw, b.reshape(1, d), W, att_s.reshape(1, d),
      att_d.reshape(1, d))


def _tc_fin_body(s_ref, den_ref, h_ref, selfw_ref, b_ref, o_ref):
    o_ref[...] = _combine(s_ref, den_ref, h_ref, selfw_ref, b_ref)


def _tc_fin(S, den, h, selfw, b, blk=1000):
    n, d = h.shape
    return pl.pallas_call(
        _tc_fin_body,
        grid=(n // blk,),
        in_specs=[
            pl.BlockSpec((_NC, blk, d), lambda i: (0, i, 0)),
            pl.BlockSpec((_NW, blk, 1), lambda i: (0, i, 0)),
            pl.BlockSpec((blk, d), lambda i: (i, 0)),
            pl.BlockSpec((blk, 1), lambda i: (i, 0)),
            pl.BlockSpec((1, d), lambda i: (0, 0)),
        ],
        out_specs=pl.BlockSpec((blk, d), lambda i: (i, 0)),
        out_shape=jax.ShapeDtypeStruct((n, d), jnp.float32),
    )(S, den, h, selfw, b.reshape(1, d))


def _sc_edge_aggregate(h, asrc_p, adst_p, src2, dst2, srows, ct0, ct1):
    n, d = h.shape
    apad = asrc_p.shape[0]
    rpt = srows // _NS
    zfull, zrem = divmod(rpt, _CHUNK)
    mesh = plsc.VectorSubcoreMesh(core_axis_name="c", subcore_axis_name="s")
    cp = pltpu.CompilerParams()
    if "needs_layout_passes" in pltpu.CompilerParams.__dataclass_fields__:
        cp = dataclasses.replace(cp, needs_layout_passes=False)

    assert ct0 % _CB == 0 and ct1 % _CB == 0 and (_CB % 2) == 0
    assert ct0 >= 2 * _CB and ct1 >= 2 * _CB

    @functools.partial(
        pl.kernel,
        out_type=[
            jax.ShapeDtypeStruct((_NC, srows, d), jnp.float32),
            jax.ShapeDtypeStruct((_NW, apad), jnp.float32),
        ],
        mesh=mesh,
        compiler_params=cp,
        scratch_types=[
            pltpu.VMEM_SHARED((srows, d), jnp.float32),
            pltpu.VMEM((apad,), jnp.float32),
            pltpu.VMEM((apad,), jnp.float32),
            pltpu.VMEM((apad,), jnp.float32),
            pltpu.VMEM((2, _CB, _CHUNK), jnp.int32),
            pltpu.VMEM((2, _CB, _CHUNK), jnp.int32),
            pltpu.VMEM((_CHUNK,), jnp.float32),
            pltpu.VMEM((2, _CHUNK, d), jnp.float32),
            pltpu.SemaphoreType.DMA,
            pltpu.SemaphoreType.DMA,
            pltpu.SemaphoreType.DMA,
            pltpu.SemaphoreType.DMA,
            pltpu.SemaphoreType.DMA,
        ],
    )
    def sc_kernel(h_hbm, asrc_hbm, adst_hbm, src_hbm, dst_hbm,
                  num_hbm, den_hbm,
                  acc_sh, asrc_v, adst_v, denom_v, sidx_v, didx_v,
                  w_v, rows_v,
                  sem_g0, sem_g1, sem_s0, sem_s1, sem_i):
        c = lax.axis_index("c")
        s = lax.axis_index("s")
        wid = c * _NS + s
        ct = jnp.where(c == 0, ct0, ct1)
        nbm = ct // _CB
        row0 = jnp.where(c == 0, s * ct0, _NS * ct0 + s * ct1)
        sem_g = (sem_g0, sem_g1)
        sem_s = (sem_s0, sem_s1)
        pltpu.sync_copy(asrc_hbm, asrc_v)
        pltpu.sync_copy(adst_hbm, adst_v)

        zero = jnp.zeros((_LANES,), jnp.float32)
        lane = lax.iota(jnp.int32, _LANES)

        @pl.loop(0, apad // _LANES)
        def _(v):
            denom_v[pl.ds(v * _LANES, _LANES)] = zero

        @pl.loop(0, _CHUNK)
        def _(r):
            for k2 in range(d // _LANES):
                rows_v[0, r, pl.ds(k2 * _LANES, _LANES)] = zero

        @pl.loop(0, zfull)
        def _(z):
            base = s * rpt + z * _CHUNK
            pltpu.sync_copy(rows_v.at[0], acc_sh.at[pl.ds(base, _CHUNK)])

        if zrem:
            pltpu.sync_copy(
                rows_v.at[0, pl.ds(0, zrem)],
                acc_sh.at[pl.ds(s * rpt + zfull * _CHUNK, zrem)])

        plsc.subcore_barrier()

        def attention_weights(bsel, jj):
            for i in range(_CHUNK // _LANES):
                sidx = sidx_v[bsel, jj, pl.ds(i * _LANES, _LANES)]
                didx = didx_v[bsel, jj, pl.ds(i * _LANES, _LANES)]
                e = (plsc.load_gather(asrc_v, [sidx])
                     + plsc.load_gather(adst_v, [didx]))
                w = jnp.exp(jnp.maximum(e, e * 0.2))
                w_v[pl.ds(i * _LANES, _LANES)] = w
                dsort, wsort = plsc.sort_key_val(didx, w)
                cs = plsc.cumsum(wsort)
                prev_d = dsort.at[jnp.maximum(lane - 1, 0)].get(
                    mode="promise_in_bounds")
                next_d = dsort.at[jnp.minimum(lane + 1, _LANES - 1)].get(
                    mode="promise_in_bounds")
                is_head = (lane == 0) | (dsort != prev_d)
                is_tail = (lane == _LANES - 1) | (dsort != next_d)
                base = plsc.cummax(jnp.where(is_head, cs - wsort, -1.0))
                plsc.addupdate_scatter(denom_v, [dsort], cs - base,
                                       mask=is_tail)

        def chunk_body(jb, jj, par, la_bsel, la_jj, bsel):
            j = jb * _CB + jj
            rows_cur = rows_v.at[par]
            rows_nxt = rows_v.at[1 - par]

            @pl.when(j > 0)
            def _():
                pltpu.make_async_copy(h_hbm.at[pl.ds(0, _CHUNK)],
                                      rows_nxt, sem_s[1 - par]).wait()

            @pl.when(j + 1 < ct)
            def _():
                pltpu.async_copy(h_hbm.at[sidx_v.at[la_bsel, la_jj]],
                                 rows_nxt, sem_g[1 - par])

            attention_weights(bsel, jj)

            pltpu.make_async_copy(h_hbm.at[pl.ds(0, _CHUNK)],
                                  rows_cur, sem_g[par]).wait()

            @plsc.parallel_loop(0, _CHUNK, unroll=4)
            def _(r):
                wv = plsc.load_gather(w_v, [jnp.broadcast_to(r, (_LANES,))])
                for k2 in range(d // _LANES):
                    sl = pl.ds(k2 * _LANES, _LANES)
                    rows_cur[r, sl] = rows_cur[r, sl] * wv

            pltpu.async_copy(rows_cur, acc_sh.at[didx_v.at[bsel, jj]],
                             sem_s[par], add=True)

        def wait_idx_block():
            pltpu.make_async_copy(src_hbm.at[pl.ds(0, _CB)],
                                  sidx_v.at[0], sem_i).wait()
            pltpu.make_async_copy(dst_hbm.at[pl.ds(0, _CB)],
                                  didx_v.at[0], sem_i).wait()

        pltpu.sync_copy(src_hbm.at[pl.ds(row0, _CB)], sidx_v.at[0])
        pltpu.sync_copy(dst_hbm.at[pl.ds(row0, _CB)], didx_v.at[0])
        pltpu.async_copy(src_hbm.at[pl.ds(row0 + _CB, _CB)], sidx_v.at[1],
                         sem_i)
        pltpu.async_copy(dst_hbm.at[pl.ds(row0 + _CB, _CB)], didx_v.at[1],
                         sem_i)
        pltpu.async_copy(h_hbm.at[sidx_v.at[0, 0]], rows_v.at[0], sem_g0)

        @pl.loop(0, nbm)
        def _(jb):
            bsel = lax.rem(jb, 2)

            @pl.loop(0, (_CB - 2) // 2)
            def _(pp):
                chunk_body(jb, 2 * pp, 0, bsel, 2 * pp + 1, bsel)
                chunk_body(jb, 2 * pp + 1, 1, bsel, 2 * pp + 2, bsel)

            chunk_body(jb, _CB - 2, 0, bsel, _CB - 1, bsel)
            wait_idx_block()
            chunk_body(jb, _CB - 1, 1, 1 - bsel, 0, bsel)
            nxt = jnp.minimum(jb + 2, nbm - 1)
            pltpu.async_copy(src_hbm.at[pl.ds(row0 + nxt * _CB, _CB)],
                             sidx_v.at[bsel], sem_i)
            pltpu.async_copy(dst_hbm.at[pl.ds(row0 + nxt * _CB, _CB)],
                             didx_v.at[bsel], sem_i)

        pltpu.make_async_copy(h_hbm.at[pl.ds(0, _CHUNK)], rows_v.at[1],
                              sem_s1).wait()
        wait_idx_block()

        pltpu.sync_copy(denom_v, den_hbm.at[wid])
        plsc.subcore_barrier()

        pltpu.sync_copy(acc_sh.at[pl.ds(s * rpt, rpt)],
                        num_hbm.at[c, pl.ds(s * rpt, rpt)])

    return sc_kernel(h, asrc_p, adst_p, src2, dst2)


def kernel(x, edge_index, W1, as1, ad1, b1, W2, as2, ad2, b2):
    n, d = x.shape
    e = edge_index.shape[1]
    tpt = -(-e // (_NS * _CHUNK))
    tpt = ((tpt + _CB - 1) // _CB) * _CB
    ct0 = int(_CB * round(_CORE0_FRAC * tpt / _CB))
    ct0 = max(2 * _CB, min(tpt - 2 * _CB, ct0))
    ct1 = tpt - ct0
    ep = _NS * tpt * _CHUNK
    totch = _NS * tpt
    srows = ((n + 1 + _NS * _CHUNK - 1) // (_NS * _CHUNK)) * (_NS * _CHUNK)
    apad = ((n + 1 + _LANES - 1) // _LANES) * _LANES

    src = edge_index[0]
    dst = edge_index[1]
    srcp = jnp.concatenate(
        [src, jnp.zeros((ep - e,), jnp.int32)]).reshape(totch, _CHUNK)
    dstp = jnp.concatenate(
        [dst, jnp.full((ep - e,), n, jnp.int32)]).reshape(totch, _CHUNK)

    h1, a1s, a1d, sw1 = _tc_pre(x, W1, as1, ad1)
    a1sp = jnp.pad(a1s[:, 0], (0, apad - n))
    a1dp = jnp.pad(a1d[:, 0], (0, apad - n))
    S1, den1 = _sc_edge_aggregate(h1, a1sp, a1dp, srcp, dstp, srows, ct0, ct1)

    h2, a2s, a2d, sw2 = _tc_mid(S1, den1.reshape(_NW, apad, 1), h1, sw1, b1,
                                W2, as2, ad2)
    a2sp = jnp.pad(a2s[:, 0], (0, apad - n))
    a2dp = jnp.pad(a2d[:, 0], (0, apad - n))
    S2, den2 = _sc_edge_aggregate(h2, a2sp, a2dp, srcp, dstp, srows, ct0, ct1)

    return _tc_fin(S2, den2.reshape(_NW, apad, 1), h2, sw2, b2)


# speedup vs baseline: 1.0543x; 1.0161x over previous
import dataclasses
import functools

import jax
import jax.numpy as jnp
from jax import lax
from jax.experimental import pallas as pl
from jax.experimental.pallas import tpu as pltpu
from jax.experimental.pallas import tpu_sc as plsc

_LANES = 16
_NC, _NS = 2, 16
_NW = _NC * _NS
_CHUNK = 48
_CB = 4
_CORE0_FRAC = 0.62


def _tc_pre_body(x_ref, w_ref, as_ref, ad_ref, h_ref, asrc_ref, adst_ref,
                 selfw_ref):
    h = jnp.dot(x_ref[...], w_ref[...], preferred_element_type=jnp.float32)
    h_ref[...] = h
    a_s = jnp.sum(h * as_ref[...], axis=1, keepdims=True)
    a_d = jnp.sum(h * ad_ref[...], axis=1, keepdims=True)
    asrc_ref[...] = a_s
    adst_ref[...] = a_d
    e = a_s + a_d
    selfw_ref[...] = jnp.exp(jnp.maximum(e, 0.2 * e))


def _tc_pre(x, W, att_s, att_d, blk=1000):
    n, d = x.shape
    return pl.pallas_call(
        _tc_pre_body,
        grid=(n // blk,),
        in_specs=[
            pl.BlockSpec((blk, d), lambda i: (i, 0)),
            pl.BlockSpec((d, d), lambda i: (0, 0)),
            pl.BlockSpec((1, d), lambda i: (0, 0)),
            pl.BlockSpec((1, d), lambda i: (0, 0)),
        ],
        out_specs=[
            pl.BlockSpec((blk, d), lambda i: (i, 0)),
            pl.BlockSpec((blk, 1), lambda i: (i, 0)),
            pl.BlockSpec((blk, 1), lambda i: (i, 0)),
            pl.BlockSpec((blk, 1), lambda i: (i, 0)),
        ],
        out_shape=[
            jax.ShapeDtypeStruct((n, d), jnp.float32),
            jax.ShapeDtypeStruct((n, 1), jnp.float32),
            jax.ShapeDtypeStruct((n, 1), jnp.float32),
            jax.ShapeDtypeStruct((n, 1), jnp.float32),
        ],
    )(x, W, att_s.reshape(1, d), att_d.reshape(1, d))


def _combine(s_ref, den_ref, h_ref, selfw_ref, b_ref):
    num = s_ref[0] + s_ref[1] + selfw_ref[...] * h_ref[...]
    den = jnp.sum(den_ref[...], axis=0) + selfw_ref[...] + 1e-16
    return jnp.maximum(num / den + b_ref[...], 0.0)


def _tc_mid_body(s_ref, den_ref, h_ref, selfw_ref, b_ref, w_ref, as_ref,
                 ad_ref, h2_ref, asrc_ref, adst_ref, selfw2_ref):
    x2 = _combine(s_ref, den_ref, h_ref, selfw_ref, b_ref)
    h2 = jnp.dot(x2, w_ref[...], preferred_element_type=jnp.float32)
    h2_ref[...] = h2
    a_s = jnp.sum(h2 * as_ref[...], axis=1, keepdims=True)
    a_d = jnp.sum(h2 * ad_ref[...], axis=1, keepdims=True)
    asrc_ref[...] = a_s
    adst_ref[...] = a_d
    e = a_s + a_d
    selfw2_ref[...] = jnp.exp(jnp.maximum(e, 0.2 * e))


def _tc_mid(S, den, h, selfw, b, W, att_s, att_d, blk=1000):
    n, d = h.shape
    return pl.pallas_call(
        _tc_mid_body,
        grid=(n // blk,),
        in_specs=[
            pl.BlockSpec((_NC, blk, d), lambda i: (0, i, 0)),
            pl.BlockSpec((_NW, blk, 1), lambda i: (0, i, 0)),
            pl.BlockSpec((blk, d), lambda i: (i, 0)),
            pl.BlockSpec((blk, 1), lambda i: (i, 0)),
            pl.BlockSpec((1, d), lambda i: (0, 0)),
            pl.BlockSpec((d, d), lambda i: (0, 0)),
            pl.BlockSpec((1, d), lambda i: (0, 0)),
            pl.BlockSpec((1, d), lambda i: (0, 0)),
        ],
        out_specs=[
            pl.BlockSpec((blk, d), lambda i: (i, 0)),
            pl.BlockSpec((blk, 1), lambda i: (i, 0)),
            pl.BlockSpec((blk, 1), lambda i: (i, 0)),
            pl.BlockSpec((blk, 1), lambda i: (i, 0)),
        ],
        out_shape=[
            jax.ShapeDtypeStruct((n, d), jnp.float32),
            jax.ShapeDtypeStruct((n, 1), jnp.float32),
            jax.ShapeDtypeStruct((n, 1), jnp.float32),
            jax.ShapeDtypeStruct((n, 1), jnp.float32),
        ],
    )(S, den, h, selfw, b.reshape(1, d), W, att_s.reshape(1, d),
      att_d.reshape(1, d))


def _tc_fin_body(s_ref, den_ref, h_ref, selfw_ref, b_ref, o_ref):
    o_ref[...] = _combine(s_ref, den_ref, h_ref, selfw_ref, b_ref)


def _tc_fin(S, den, h, selfw, b, blk=1000):
    n, d = h.shape
    return pl.pallas_call(
        _tc_fin_body,
        grid=(n // blk,),
        in_specs=[
            pl.BlockSpec((_NC, blk, d), lambda i: (0, i, 0)),
            pl.BlockSpec((_NW, blk, 1), lambda i: (0, i, 0)),
            pl.BlockSpec((blk, d), lambda i: (i, 0)),
            pl.BlockSpec((blk, 1), lambda i: (i, 0)),
            pl.BlockSpec((1, d), lambda i: (0, 0)),
        ],
        out_specs=pl.BlockSpec((blk, d), lambda i: (i, 0)),
        out_shape=jax.ShapeDtypeStruct((n, d), jnp.float32),
    )(S, den, h, selfw, b.reshape(1, d))


def _sc_edge_aggregate(h, asrc_p, adst_p, src2, dst2, srows, ct0, ct1):
    n, d = h.shape
    apad = asrc_p.shape[0]
    rpt = srows // _NS
    zfull, zrem = divmod(rpt, _CHUNK)
    mesh = plsc.VectorSubcoreMesh(core_axis_name="c", subcore_axis_name="s")
    cp = pltpu.CompilerParams()
    if "needs_layout_passes" in pltpu.CompilerParams.__dataclass_fields__:
        cp = dataclasses.replace(cp, needs_layout_passes=False)

    assert ct0 % _CB == 0 and ct1 % _CB == 0 and (_CB % 2) == 0
    assert ct0 >= 2 * _CB and ct1 >= 2 * _CB

    @functools.partial(
        pl.kernel,
        out_type=[
            jax.ShapeDtypeStruct((_NC, srows, d), jnp.float32),
            jax.ShapeDtypeStruct((_NW, apad), jnp.float32),
        ],
        mesh=mesh,
        compiler_params=cp,
        scratch_types=[
            pltpu.VMEM_SHARED((srows, d), jnp.float32),
            pltpu.VMEM((apad,), jnp.float32),
            pltpu.VMEM((apad,), jnp.float32),
            pltpu.VMEM((apad,), jnp.float32),
            pltpu.VMEM((2, _CB, _CHUNK), jnp.int32),
            pltpu.VMEM((2, _CB, _CHUNK), jnp.int32),
            pltpu.VMEM((_CHUNK,), jnp.float32),
            pltpu.VMEM((2, _CHUNK, d), jnp.float32),
            pltpu.SemaphoreType.DMA,
            pltpu.SemaphoreType.DMA,
            pltpu.SemaphoreType.DMA,
            pltpu.SemaphoreType.DMA,
            pltpu.SemaphoreType.DMA,
        ],
    )
    def sc_kernel(h_hbm, asrc_hbm, adst_hbm, src_hbm, dst_hbm,
                  num_hbm, den_hbm,
                  acc_sh, asrc_v, adst_v, denom_v, sidx_v, didx_v,
                  w_v, rows_v,
                  sem_g0, sem_g1, sem_s0, sem_s1, sem_i):
        c = lax.axis_index("c")
        s = lax.axis_index("s")
        wid = c * _NS + s
        ct = jnp.where(c == 0, ct0, ct1)
        nbm = ct // _CB
        row0 = jnp.where(c == 0, s * ct0, _NS * ct0 + s * ct1)
        sem_g = (sem_g0, sem_g1)
        sem_s = (sem_s0, sem_s1)
        pltpu.sync_copy(asrc_hbm, asrc_v)
        pltpu.sync_copy(adst_hbm, adst_v)

        zero = jnp.zeros((_LANES,), jnp.float32)
        lane = lax.iota(jnp.int32, _LANES)

        @pl.loop(0, apad // _LANES)
        def _(v):
            denom_v[pl.ds(v * _LANES, _LANES)] = zero

        @pl.loop(0, _CHUNK)
        def _(r):
            for k2 in range(d // _LANES):
                rows_v[0, r, pl.ds(k2 * _LANES, _LANES)] = zero

        @pl.loop(0, zfull)
        def _(z):
            base = s * rpt + z * _CHUNK
            pltpu.sync_copy(rows_v.at[0], acc_sh.at[pl.ds(base, _CHUNK)])

        if zrem:
            pltpu.sync_copy(
                rows_v.at[0, pl.ds(0, zrem)],
                acc_sh.at[pl.ds(s * rpt + zfull * _CHUNK, zrem)])

        plsc.subcore_barrier()

        def attention_weights(bsel, jj):
            for i in range(_CHUNK // _LANES):
                sidx = sidx_v[bsel, jj, pl.ds(i * _LANES, _LANES)]
                didx = didx_v[bsel, jj, pl.ds(i * _LANES, _LANES)]
                e = (plsc.load_gather(asrc_v, [sidx])
                     + plsc.load_gather(adst_v, [didx]))
                w = jnp.exp(jnp.maximum(e, e * 0.2))
                w_v[pl.ds(i * _LANES, _LANES)] = w
                dsort, wsort = plsc.sort_key_val(didx, w)
                cs = plsc.cumsum(wsort)
                prev_d = dsort.at[jnp.maximum(lane - 1, 0)].get(
                    mode="promise_in_bounds")
                next_d = dsort.at[jnp.minimum(lane + 1, _LANES - 1)].get(
                    mode="promise_in_bounds")
                is_head = (lane == 0) | (dsort != prev_d)
                is_tail = (lane == _LANES - 1) | (dsort != next_d)
                base = plsc.cummax(jnp.where(is_head, cs - wsort, -1.0))
                plsc.addupdate_scatter(denom_v, [dsort], cs - base,
                                       mask=is_tail)

        def chunk_body(jb, jj, par, la_bsel, la_jj, bsel):
            j = jb * _CB + jj
            rows_cur = rows_v.at[par]
            rows_nxt = rows_v.at[1 - par]

            @pl.when(j > 0)
            def _():
                pltpu.make_async_copy(h_hbm.at[pl.ds(0, _CHUNK)],
                                      rows_nxt, sem_s[1 - par]).wait()

            @pl.when(j + 1 < ct)
            def _():
                pltpu.async_copy(h_hbm.at[sidx_v.at[la_bsel, la_jj]],
                                 rows_nxt, sem_g[1 - par])

            attention_weights(bsel, jj)

            pltpu.make_async_copy(h_hbm.at[pl.ds(0, _CHUNK)],
                                  rows_cur, sem_g[par]).wait()

            @plsc.parallel_loop(0, _CHUNK, unroll=4)
            def _(r):
                wv = plsc.load_gather(w_v, [jnp.broadcast_to(r, (_LANES,))])
                for k2 in range(d // _LANES):
                    sl = pl.ds(k2 * _LANES, _LANES)
                    rows_cur[r, sl] = rows_cur[r, sl] * wv

            pltpu.async_copy(rows_cur, acc_sh.at[didx_v.at[bsel, jj]],
                             sem_s[par], add=True)

        def wait_idx_block():
            pltpu.make_async_copy(src_hbm.at[pl.ds(0, _CB)],
                                  sidx_v.at[0], sem_i).wait()
            pltpu.make_async_copy(dst_hbm.at[pl.ds(0, _CB)],
                                  didx_v.at[0], sem_i).wait()

        pltpu.sync_copy(src_hbm.at[pl.ds(row0, _CB)], sidx_v.at[0])
        pltpu.sync_copy(dst_hbm.at[pl.ds(row0, _CB)], didx_v.at[0])
        pltpu.async_copy(src_hbm.at[pl.ds(row0 + _CB, _CB)], sidx_v.at[1],
                         sem_i)
        pltpu.async_copy(dst_hbm.at[pl.ds(row0 + _CB, _CB)], didx_v.at[1],
                         sem_i)
        pltpu.async_copy(h_hbm.at[sidx_v.at[0, 0]], rows_v.at[0], sem_g0)

        @pl.loop(0, nbm)
        def _(jb):
            bsel = lax.rem(jb, 2)

            @pl.loop(0, (_CB - 2) // 2)
            def _(pp):
                chunk_body(jb, 2 * pp, 0, bsel, 2 * pp + 1, bsel)
                chunk_body(jb, 2 * pp + 1, 1, bsel, 2 * pp + 2, bsel)

            chunk_body(jb, _CB - 2, 0, bsel, _CB - 1, bsel)
            wait_idx_block()
            chunk_body(jb, _CB - 1, 1, 1 - bsel, 0, bsel)
            nxt = jnp.minimum(jb + 2, nbm - 1)
            pltpu.async_copy(src_hbm.at[pl.ds(row0 + nxt * _CB, _CB)],
                             sidx_v.at[bsel], sem_i)
            pltpu.async_copy(dst_hbm.at[pl.ds(row0 + nxt * _CB, _CB)],
                             didx_v.at[bsel], sem_i)

        pltpu.make_async_copy(h_hbm.at[pl.ds(0, _CHUNK)], rows_v.at[1],
                              sem_s1).wait()
        wait_idx_block()

        pltpu.sync_copy(denom_v, den_hbm.at[wid])
        plsc.subcore_barrier()

        pltpu.sync_copy(acc_sh.at[pl.ds(s * rpt, rpt)],
                        num_hbm.at[c, pl.ds(s * rpt, rpt)])

    return sc_kernel(h, asrc_p, adst_p, src2, dst2)


def kernel(x, edge_index, W1, as1, ad1, b1, W2, as2, ad2, b2):
    n, d = x.shape
    e = edge_index.shape[1]
    tpt = -(-e // (_NS * _CHUNK))
    tpt = ((tpt + _CB - 1) // _CB) * _CB
    ct0 = int(_CB * round(_CORE0_FRAC * tpt / _CB))
    ct0 = max(2 * _CB, min(tpt - 2 * _CB, ct0))
    ct1 = tpt - ct0
    ep = _NS * tpt * _CHUNK
    totch = _NS * tpt
    srows = ((n + 1 + _NS * _CHUNK - 1) // (_NS * _CHUNK)) * (_NS * _CHUNK)
    apad = ((n + 1 + _LANES - 1) // _LANES) * _LANES

    src = edge_index[0]
    dst = edge_index[1]
    srcp = jnp.concatenate(
        [src, jnp.zeros((ep - e,), jnp.int32)]).reshape(totch, _CHUNK)
    dstp = jnp.concatenate(
        [dst, jnp.full((ep - e,), n, jnp.int32)]).reshape(totch, _CHUNK)

    h1, a1s, a1d, sw1 = _tc_pre(x, W1, as1, ad1)
    a1sp = jnp.pad(a1s[:, 0], (0, apad - n))
    a1dp = jnp.pad(a1d[:, 0], (0, apad - n))
    S1, den1 = _sc_edge_aggregate(h1, a1sp, a1dp, srcp, dstp, srows, ct0, ct1)

    h2, a2s, a2d, sw2 = _tc_mid(S1, den1.reshape(_NW, apad, 1), h1, sw1, b1,
                                W2, as2, ad2)
    a2sp = jnp.pad(a2s[:, 0], (0, apad - n))
    a2dp = jnp.pad(a2d[:, 0], (0, apad - n))
    S2, den2 = _sc_edge_aggregate(h2, a2sp, a2dp, srcp, dstp, srows, ct0, ct1)

    return _tc_fin(S2, den2.reshape(_NW, apad, 1), h2, sw2, b2)
